# baseline (speedup 1.0000x reference)
import functools

import jax
import jax.numpy as jnp
from jax import lax
from jax.experimental import pallas as pl
from jax.experimental.pallas import tpu as pltpu
from jax.experimental.pallas import tpu_sc as plsc

_HIDDEN = 64
_GRP = 128
_NC, _NS = 2, 16
_NW = _NC * _NS
_L = 16


@functools.cache
def _make_packer(vocab):
    full_cols = vocab // _GRP
    tail = vocab - full_cols * _GRP
    assert tail == 64 and full_cols % 2 == 0
    packed_rows = full_cols * 64 + tail // 2
    mesh = plsc.VectorSubcoreMesh(core_axis_name="c", subcore_axis_name="s")
    pairs = full_cols // 2
    nk = pairs // _NW
    rem = pairs - nk * _NW
    assert nk % 2 == 0

    @functools.partial(
        pl.kernel,
        out_type=jax.ShapeDtypeStruct((packed_rows, _GRP), jnp.float32),
        mesh=mesh,
        scratch_types=[
            pltpu.VMEM((_HIDDEN, 2 * _GRP), jnp.float32),
            pltpu.VMEM((_HIDDEN, 2 * _GRP), jnp.float32),
            pltpu.VMEM((2 * _HIDDEN, _GRP), jnp.float32),
            pltpu.VMEM((2 * _HIDDEN, _GRP), jnp.float32),
            pltpu.VMEM((32, _GRP), jnp.float32),
            pltpu.SemaphoreType.DMA,
            pltpu.SemaphoreType.DMA,
            pltpu.SemaphoreType.DMA,
            pltpu.SemaphoreType.DMA,
        ],
        compiler_params=pltpu.CompilerParams(needs_layout_passes=False),
    )
    def packer(tt_hbm, tail_hbm, out_hbm,
               in0, in1, to0, to1, tlb, li0, li1, so0, so1):
        w = lax.axis_index("s") * _NC + lax.axis_index("c")
        inb = (in0, in1)
        tob = (to0, to1)
        lsem = (li0, li1)
        ssem = (so0, so1)

        def col_of(k):
            return k * _NW + w

        def fire(k, b):
            pltpu.async_copy(
                tt_hbm.at[:, pl.ds(col_of(k) * 2 * _GRP, 2 * _GRP)],
                inb[b], lsem[b],
            )

        def wait_load(b):
            pltpu.make_async_copy(
                tt_hbm.at[:, pl.ds(0, 2 * _GRP)], inb[b], lsem[b]
            ).wait()

        def wait_store(b):
            pltpu.make_async_copy(
                tob[b], out_hbm.at[pl.ds(0, 2 * _HIDDEN)], ssem[b]
            ).wait()

        iota16 = lax.iota(jnp.int32, _L)
        rows_c = [h0 + iota16 for h0 in range(0, _HIDDEN, _L)]
        cps = [(iota16 + k) & 15 for k in range(_L)]

        def transpose_store(k, b):
            src = inb[b]
            dst = tob[b]

            @plsc.parallel_loop(0, 16)
            def _(vb):
                vl0 = vb * _L
                for hb in range(4):
                    hrow = rows_c[hb]
                    for kk in range(_L):
                        vlv = vl0 + cps[kk]
                        vec = plsc.load_gather(src, [hrow, vlv])
                        qv = lax.shift_right_logical(vlv, 1)
                        colv = lax.shift_left(vlv & 1, 6) + hrow
                        plsc.store_scatter(dst, [qv, colv], vec)

            pltpu.async_copy(
                tob[b],
                out_hbm.at[pl.ds(col_of(k) * 2 * _HIDDEN, 2 * _HIDDEN)],
                ssem[b],
            )

        @pl.when(w == rem)
        def _():
            pltpu.sync_copy(tail_hbm, tlb)
            pltpu.sync_copy(tlb, out_hbm.at[pl.ds(full_cols * 64, 32)])

        fire(0, 0)
        fire(1, 1)

        @pl.loop(0, nk, step=2)
        def _(k):
            wait_load(0)

            @pl.when(k >= 2)
            def _():
                wait_store(0)

            transpose_store(k, 0)

            @pl.when((k + 2 < nk) | ((k + 2 == nk) & (w < rem)))
            def _():
                fire(k + 2, 0)

            wait_load(1)

            @pl.when(k >= 2)
            def _():
                wait_store(1)

            transpose_store(k + 1, 1)

            @pl.when(k + 3 < nk)
            def _():
                fire(k + 3, 1)

        @pl.when(w < rem)
        def _():
            wait_load(0)
            wait_store(0)
            transpose_store(nk, 0)

        wait_store(0)
        wait_store(1)

    return packer


@functools.cache
def _make_gather(bs, seq):
    nbt = bs // _GRP
    assert nbt == _NW
    nchunks = seq // 4
    assert nchunks * 4 == seq and nchunks % 2 == 0
    mesh = plsc.VectorSubcoreMesh(core_axis_name="c", subcore_axis_name="s")

    @functools.partial(
        pl.kernel,
        out_type=jax.ShapeDtypeStruct(
            (seq, _HIDDEN // 8, nbt, 8, _GRP), jnp.float32
        ),
        mesh=mesh,
        scratch_types=[
            pltpu.VMEM((4, _GRP), jnp.int32),
            pltpu.VMEM((4, _GRP), jnp.int32),
            pltpu.VMEM((4, _GRP, _HIDDEN), jnp.float32),
            pltpu.VMEM((4, _GRP, _HIDDEN), jnp.float32),
            pltpu.VMEM((2, _HIDDEN // 8, 8, _GRP), jnp.float32),
            pltpu.VMEM((2, _HIDDEN // 8, 8, _GRP), jnp.float32),
            pltpu.SemaphoreType.DMA,
            pltpu.SemaphoreType.DMA,
            pltpu.SemaphoreType.DMA,
            pltpu.SemaphoreType.DMA,
            pltpu.SemaphoreType.DMA,
            pltpu.SemaphoreType.DMA,
        ],
        compiler_params=pltpu.CompilerParams(
            use_tc_tiling_on_sc=False, needs_layout_passes=False
        ),
    )
    def gather(table_hbm, ids_hbm, out_hbm,
               idx0, idx1, rows0, rows1, tp0, tp1,
               i0, i1, g0, g1, st0, st1):
        w = lax.axis_index("s") * _NC + lax.axis_index("c")
        idx_v = (idx0, idx1)
        rows_v = (rows0, rows1)
        tpb = (tp0, tp1)
        isem = (i0, i1)
        gsem = (g0, g1)
        stsem = (st0, st1)

        def fire_idx(c, b):
            pltpu.async_copy(
                ids_hbm.at[c // 2, w, pl.ds((4 * c) % 8, 4)],
                idx_v[b], isem[b],
            )

        def wait_idx(b):
            pltpu.make_async_copy(
                ids_hbm.at[0, 0, pl.ds(0, 4)], idx_v[b], isem[b]
            ).wait()

        def fire_gathers(b):
            for j in range(4):
                pltpu.async_copy(
                    table_hbm.at[idx_v[b].at[j]], rows_v[b].at[j], gsem[b]
                )

        def wait_gathers(b):
            for j in range(4):
                pltpu.make_async_copy(
                    table_hbm.at[pl.ds(0, _GRP)], rows_v[b].at[j], gsem[b]
                ).wait()

        def wait_pair_store(p):
            for u in range(2):
                pltpu.make_async_copy(
                    tpb[p].at[u], out_hbm.at[0, :, w], stsem[p]
                ).wait()

        iota16 = lax.iota(jnp.int32, _L)
        rows_c = [c0 * _L + iota16 for c0 in range(8)]
        cps = [(iota16 + k) & 15 for k in range(_L)]

        def transpose_pair(src4, j2, dst):
            @pl.loop(0, 2)
            def _(u):
                src = src4.at[2 * j2 + u]
                d = dst.at[u]

                @plsc.parallel_loop(0, 8)
                def _(tb8):
                    trow = tb8 * _L + iota16
                    for hb in range(4):
                        h0 = hb * _L
                        for kk in range(_L):
                            hv = h0 + cps[kk]
                            vec = plsc.load_gather(src, [trow, hv])
                            plsc.store_scatter(
                                d,
                                [lax.shift_right_logical(hv, 3), hv & 7,
                                 trow],
                                vec,
                            )

        def transpose_store(c, b):
            for p in range(2):
                if b == 0:
                    @pl.when(c > 0)
                    def _():
                        wait_pair_store(p)
                else:
                    wait_pair_store(p)
                transpose_pair(rows_v[b], p, tpb[p])
                s0 = 4 * c + 2 * p
                for u in range(2):
                    pltpu.async_copy(
                        tpb[p].at[u], out_hbm.at[s0 + u, :, w], stsem[p]
                    )

        fire_idx(0, 0)
        wait_idx(0)
        fire_gathers(0)
        fire_idx(1, 1)

        @pl.loop(0, nchunks, step=2)
        def _(c):
            wait_idx(1)
            fire_gathers(1)
            wait_gathers(0)

            @pl.when(c + 2 < nchunks)
            def _():
                fire_idx(c + 2, 0)

            transpose_store(c, 0)

            @pl.when(c + 2 < nchunks)
            def _():
                wait_idx(0)
                fire_gathers(0)

            wait_gathers(1)

            @pl.when(c + 3 < nchunks)
            def _():
                fire_idx(c + 3, 1)

            transpose_store(c + 1, 1)

        wait_pair_store(0)
        wait_pair_store(1)

    return gather


def _ws_body(ws_ref, out_ref):
    out_ref[...] = jnp.broadcast_to(ws_ref[...], out_ref.shape)


@functools.cache
def _make_ws_broadcast(bs, w):
    blk = 256
    assert bs % blk == 0
    return pl.pallas_call(
        _ws_body,
        grid=(bs // blk,),
        in_specs=[pl.BlockSpec((1, w, _HIDDEN), lambda i: (0, 0, 0))],
        out_specs=pl.BlockSpec((blk, w, _HIDDEN), lambda i: (i, 0, 0)),
        out_shape=jax.ShapeDtypeStruct((bs, w, _HIDDEN), jnp.float32),
    )


def kernel(input_ids, attention_mask, init_workspace, emb_table):
    bs, seq = input_ids.shape
    vocab = emb_table.shape[0]
    full = (vocab // _GRP) * _GRP
    tail2 = emb_table[full:].reshape((vocab - full) // 2, 2 * _HIDDEN)
    packed = _make_packer(vocab)(emb_table.T, tail2)
    table_lin = packed.reshape(packed.shape[0] * 2, _HIDDEN)
    ids4 = (
        input_ids.T.reshape(seq // 8, 8, bs // _GRP, _GRP)
        .transpose(0, 2, 1, 3)
    )
    out5 = _make_gather(bs, seq)(table_lin, ids4)
    embeddings = jnp.transpose(out5, (2, 4, 0, 1, 3)).reshape(bs, seq, _HIDDEN)
    workspace = _make_ws_broadcast(bs, init_workspace.shape[1])(init_workspace)
    return (workspace, embeddings)


# speedup vs baseline: 1.0466x; 1.0466x over previous
import functools

import jax
import jax.numpy as jnp
from jax import lax
from jax.experimental import pallas as pl
from jax.experimental.pallas import tpu as pltpu
from jax.experimental.pallas import tpu_sc as plsc

_HIDDEN = 64
_GRP = 128
_NC, _NS = 2, 16
_NW = _NC * _NS
_L = 16


@functools.cache
def _make_packer(vocab):
    full_cols = vocab // _GRP
    tail = vocab - full_cols * _GRP
    assert tail == 64
    packed_rows = full_cols * 64 + tail // 2
    mesh = plsc.VectorSubcoreMesh(core_axis_name="c", subcore_axis_name="s")
    nk = full_cols // _NW
    rem = full_cols - nk * _NW
    assert nk % 2 == 0

    @functools.partial(
        pl.kernel,
        out_type=jax.ShapeDtypeStruct((packed_rows, _GRP), jnp.float32),
        mesh=mesh,
        scratch_types=[
            pltpu.VMEM((_HIDDEN, _GRP), jnp.float32),
            pltpu.VMEM((_HIDDEN, _GRP), jnp.float32),
            pltpu.VMEM((_HIDDEN, _GRP), jnp.float32),
            pltpu.VMEM((_HIDDEN, _GRP), jnp.float32),
            pltpu.VMEM((32, _GRP), jnp.float32),
            pltpu.SemaphoreType.DMA,
            pltpu.SemaphoreType.DMA,
            pltpu.SemaphoreType.DMA,
            pltpu.SemaphoreType.DMA,
        ],
        compiler_params=pltpu.CompilerParams(
            needs_layout_passes=False, disable_bounds_checks=True
        ),
    )
    def packer(tt_hbm, tail_hbm, out_hbm,
               in0, in1, to0, to1, tlb, li0, li1, so0, so1):
        w = lax.axis_index("s") * _NC + lax.axis_index("c")
        inb = (in0, in1)
        tob = (to0, to1)
        lsem = (li0, li1)
        ssem = (so0, so1)

        def col_of(k):
            return k * _NW + w

        def fire(k, b):
            pltpu.async_copy(
                tt_hbm.at[:, pl.ds(col_of(k) * _GRP, _GRP)], inb[b], lsem[b]
            )

        def wait_load(b):
            pltpu.make_async_copy(
                tt_hbm.at[:, pl.ds(0, _GRP)], inb[b], lsem[b]
            ).wait()

        def wait_store(b):
            pltpu.make_async_copy(
                tob[b], out_hbm.at[pl.ds(0, _HIDDEN)], ssem[b]
            ).wait()

        iota16 = lax.iota(jnp.int32, _L)
        rows_c = [h0 + iota16 for h0 in range(0, _HIDDEN, _L)]
        cps = [(iota16 + k) & 15 for k in range(_L)]

        def transpose_store(k, b):
            src = inb[b]
            dst = tob[b]

            @plsc.parallel_loop(0, 8)
            def _(vb):
                vl0 = vb * _L
                for hb in range(4):
                    hrow = rows_c[hb]
                    for kk in range(_L):
                        vlv = vl0 + cps[kk]
                        vec = plsc.load_gather(src, [hrow, vlv])
                        qv = lax.shift_right_logical(vlv, 1)
                        colv = lax.shift_left(vlv & 1, 6) + hrow
                        plsc.store_scatter(dst, [qv, colv], vec)

            pltpu.async_copy(
                tob[b], out_hbm.at[pl.ds(col_of(k) * _HIDDEN, _HIDDEN)],
                ssem[b],
            )

        @pl.when(w == rem)
        def _():
            pltpu.sync_copy(tail_hbm, tlb)
            pltpu.sync_copy(tlb, out_hbm.at[pl.ds(full_cols * 64, 32)])

        fire(0, 0)
        fire(1, 1)

        @pl.loop(0, nk, step=2)
        def _(k):
            wait_load(0)

            @pl.when(k >= 2)
            def _():
                wait_store(0)

            transpose_store(k, 0)

            @pl.when((k + 2 < nk) | ((k + 2 == nk) & (w < rem)))
            def _():
                fire(k + 2, 0)

            wait_load(1)

            @pl.when(k >= 2)
            def _():
                wait_store(1)

            transpose_store(k + 1, 1)

            @pl.when(k + 3 < nk)
            def _():
                fire(k + 3, 1)

        @pl.when(w < rem)
        def _():
            wait_load(0)
            wait_store(0)
            transpose_store(nk, 0)

        wait_store(0)
        wait_store(1)

    return packer


@functools.cache
def _make_gather(bs, seq):
    nbt = bs // _GRP
    assert nbt == _NW
    nchunks = seq // 4
    assert nchunks * 4 == seq and nchunks % 2 == 0
    mesh = plsc.VectorSubcoreMesh(core_axis_name="c", subcore_axis_name="s")

    @functools.partial(
        pl.kernel,
        out_type=jax.ShapeDtypeStruct(
            (seq, _HIDDEN // 8, nbt, 8, _GRP), jnp.float32
        ),
        mesh=mesh,
        scratch_types=[
            pltpu.VMEM((4, _GRP), jnp.int32),
            pltpu.VMEM((4, _GRP), jnp.int32),
            pltpu.VMEM((4, _GRP, _HIDDEN), jnp.float32),
            pltpu.VMEM((4, _GRP, _HIDDEN), jnp.float32),
            pltpu.VMEM((2, _HIDDEN // 8, 8, _GRP), jnp.float32),
            pltpu.VMEM((2, _HIDDEN // 8, 8, _GRP), jnp.float32),
            pltpu.SemaphoreType.DMA,
            pltpu.SemaphoreType.DMA,
            pltpu.SemaphoreType.DMA,
            pltpu.SemaphoreType.DMA,
            pltpu.SemaphoreType.DMA,
            pltpu.SemaphoreType.DMA,
        ],
        compiler_params=pltpu.CompilerParams(
            use_tc_tiling_on_sc=False, needs_layout_passes=False,
            disable_bounds_checks=True,
        ),
    )
    def gather(table_hbm, ids_hbm, out_hbm,
               idx0, idx1, rows0, rows1, tp0, tp1,
               i0, i1, g0, g1, st0, st1):
        w = lax.axis_index("s") * _NC + lax.axis_index("c")
        idx_v = (idx0, idx1)
        rows_v = (rows0, rows1)
        tpb = (tp0, tp1)
        isem = (i0, i1)
        gsem = (g0, g1)
        stsem = (st0, st1)

        def fire_idx(c, b):
            pltpu.async_copy(
                ids_hbm.at[c // 2, w, pl.ds((4 * c) % 8, 4)],
                idx_v[b], isem[b],
            )

        def wait_idx(b):
            pltpu.make_async_copy(
                ids_hbm.at[0, 0, pl.ds(0, 4)], idx_v[b], isem[b]
            ).wait()

        def fire_gathers(b):
            for j in range(4):
                pltpu.async_copy(
                    table_hbm.at[idx_v[b].at[j]], rows_v[b].at[j], gsem[b]
                )

        def wait_gathers(b):
            for j in range(4):
                pltpu.make_async_copy(
                    table_hbm.at[pl.ds(0, _GRP)], rows_v[b].at[j], gsem[b]
                ).wait()

        def wait_pair_store(p):
            for u in range(2):
                pltpu.make_async_copy(
                    tpb[p].at[u], out_hbm.at[0, :, w], stsem[p]
                ).wait()

        iota16 = lax.iota(jnp.int32, _L)
        rows_c = [c0 * _L + iota16 for c0 in range(8)]
        cps = [(iota16 + k) & 15 for k in range(_L)]

        def transpose_pair(src4, j2, dst):
            @pl.loop(0, 2)
            def _(u):
                src = src4.at[2 * j2 + u]
                d = dst.at[u]

                @plsc.parallel_loop(0, 8)
                def _(tb8):
                    trow = tb8 * _L + iota16
                    for hb in range(4):
                        h0 = hb * _L
                        for kk in range(_L):
                            hv = h0 + cps[kk]
                            vec = plsc.load_gather(src, [trow, hv])
                            plsc.store_scatter(
                                d,
                                [lax.shift_right_logical(hv, 3), hv & 7,
                                 trow],
                                vec,
                            )

        def transpose_store(c, b):
            for p in range(2):
                if b == 0:
                    @pl.when(c > 0)
                    def _():
                        wait_pair_store(p)
                else:
                    wait_pair_store(p)
                transpose_pair(rows_v[b], p, tpb[p])
                s0 = 4 * c + 2 * p
                for u in range(2):
                    pltpu.async_copy(
                        tpb[p].at[u], out_hbm.at[s0 + u, :, w], stsem[p]
                    )

        fire_idx(0, 0)
        wait_idx(0)
        fire_gathers(0)
        fire_idx(1, 1)

        @pl.loop(0, nchunks, step=2)
        def _(c):
            wait_idx(1)
            fire_gathers(1)
            wait_gathers(0)

            @pl.when(c + 2 < nchunks)
            def _():
                fire_idx(c + 2, 0)

            transpose_store(c, 0)

            @pl.when(c + 2 < nchunks)
            def _():
                wait_idx(0)
                fire_gathers(0)

            wait_gathers(1)

            @pl.when(c + 3 < nchunks)
            def _():
                fire_idx(c + 3, 1)

            transpose_store(c + 1, 1)

        wait_pair_store(0)
        wait_pair_store(1)

    return gather


def _ws_body(ws_ref, out_ref):
    out_ref[...] = jnp.broadcast_to(ws_ref[...], out_ref.shape)


@functools.cache
def _make_ws_broadcast(bs, w):
    blk = 256
    assert bs % blk == 0
    return pl.pallas_call(
        _ws_body,
        grid=(bs // blk,),
        in_specs=[pl.BlockSpec((1, w, _HIDDEN), lambda i: (0, 0, 0))],
        out_specs=pl.BlockSpec((blk, w, _HIDDEN), lambda i: (i, 0, 0)),
        out_shape=jax.ShapeDtypeStruct((bs, w, _HIDDEN), jnp.float32),
    )


def kernel(input_ids, attention_mask, init_workspace, emb_table):
    bs, seq = input_ids.shape
    vocab = emb_table.shape[0]
    full = (vocab // _GRP) * _GRP
    tail2 = emb_table[full:].reshape((vocab - full) // 2, 2 * _HIDDEN)
    packed = _make_packer(vocab)(emb_table.T, tail2)
    table_lin = packed.reshape(packed.shape[0] * 2, _HIDDEN)
    ids4 = (
        input_ids.T.reshape(seq // 8, 8, bs // _GRP, _GRP)
        .transpose(0, 2, 1, 3)
    )
    out5 = _make_gather(bs, seq)(table_lin, ids4)
    embeddings = jnp.transpose(out5, (2, 4, 0, 1, 3)).reshape(bs, seq, _HIDDEN)
    workspace = _make_ws_broadcast(bs, init_workspace.shape[1])(init_workspace)
    return (workspace, embeddings)


# speedup vs baseline: 1.4321x; 1.3684x over previous
import functools

import jax
import jax.numpy as jnp
from jax import lax
from jax.experimental import pallas as pl
from jax.experimental.pallas import tpu as pltpu
from jax.experimental.pallas import tpu_sc as plsc

_HIDDEN = 64
_GRP = 128
_NC, _NS = 2, 16
_NW = _NC * _NS
_L = 16


@functools.cache
def _make_packer(vocab):
    full_cols = vocab // _GRP
    tail = vocab - full_cols * _GRP
    assert tail == 64
    packed_rows = full_cols * 64 + tail // 2
    mesh = plsc.VectorSubcoreMesh(core_axis_name="c", subcore_axis_name="s")
    nk = full_cols // _NW
    rem = full_cols - nk * _NW
    assert nk % 2 == 0

    @functools.partial(
        pl.kernel,
        out_type=jax.ShapeDtypeStruct((packed_rows, _GRP), jnp.float32),
        mesh=mesh,
        scratch_types=[
            pltpu.VMEM((_HIDDEN, _GRP), jnp.float32),
            pltpu.VMEM((_HIDDEN, _GRP), jnp.float32),
            pltpu.VMEM((_HIDDEN, _GRP), jnp.float32),
            pltpu.VMEM((_HIDDEN, _GRP), jnp.float32),
            pltpu.VMEM((32, _GRP), jnp.float32),
            pltpu.SemaphoreType.DMA,
            pltpu.SemaphoreType.DMA,
            pltpu.SemaphoreType.DMA,
            pltpu.SemaphoreType.DMA,
        ],
        compiler_params=pltpu.CompilerParams(
            needs_layout_passes=False, disable_bounds_checks=True
        ),
    )
    def packer(tt_hbm, tail_hbm, out_hbm,
               in0, in1, to0, to1, tlb, li0, li1, so0, so1):
        w = lax.axis_index("s") * _NC + lax.axis_index("c")
        inb = (in0, in1)
        tob = (to0, to1)
        lsem = (li0, li1)
        ssem = (so0, so1)

        def col_of(k):
            return k * _NW + w

        def fire(k, b):
            pltpu.async_copy(
                tt_hbm.at[:, pl.ds(col_of(k) * _GRP, _GRP)], inb[b], lsem[b]
            )

        def wait_load(b):
            pltpu.make_async_copy(
                tt_hbm.at[:, pl.ds(0, _GRP)], inb[b], lsem[b]
            ).wait()

        def wait_store(b):
            pltpu.make_async_copy(
                tob[b], out_hbm.at[pl.ds(0, _HIDDEN)], ssem[b]
            ).wait()

        iota16 = lax.iota(jnp.int32, _L)
        rows_c = [h0 + iota16 for h0 in range(0, _HIDDEN, _L)]
        cps = [(iota16 + k) & 15 for k in range(_L)]

        def transpose_store(k, b):
            src = inb[b]
            dst = tob[b]

            @plsc.parallel_loop(0, 8)
            def _(vb):
                vl0 = vb * _L
                for hb in range(4):
                    hrow = rows_c[hb]
                    vlvs = [vl0 + cps[kk] for kk in range(_L)]
                    vecs = [
                        plsc.load_gather(src, [hrow, vlvs[kk]])
                        for kk in range(_L)
                    ]
                    for kk in range(_L):
                        vlv = vlvs[kk]
                        qv = lax.shift_right_logical(vlv, 1)
                        colv = lax.shift_left(vlv & 1, 6) + hrow
                        plsc.store_scatter(dst, [qv, colv], vecs[kk])

            pltpu.async_copy(
                tob[b], out_hbm.at[pl.ds(col_of(k) * _HIDDEN, _HIDDEN)],
                ssem[b],
            )

        @pl.when(w == rem)
        def _():
            pltpu.sync_copy(tail_hbm, tlb)
            pltpu.sync_copy(tlb, out_hbm.at[pl.ds(full_cols * 64, 32)])

        fire(0, 0)
        fire(1, 1)

        @pl.loop(0, nk, step=2)
        def _(k):
            wait_load(0)

            @pl.when(k >= 2)
            def _():
                wait_store(0)

            transpose_store(k, 0)

            @pl.when((k + 2 < nk) | ((k + 2 == nk) & (w < rem)))
            def _():
                fire(k + 2, 0)

            wait_load(1)

            @pl.when(k >= 2)
            def _():
                wait_store(1)

            transpose_store(k + 1, 1)

            @pl.when(k + 3 < nk)
            def _():
                fire(k + 3, 1)

        @pl.when(w < rem)
        def _():
            wait_load(0)
            wait_store(0)
            transpose_store(nk, 0)

        wait_store(0)
        wait_store(1)

    return packer


@functools.cache
def _make_gather(bs, seq):
    nbt = bs // _GRP
    assert nbt == _NW
    nchunks = seq // 4
    assert nchunks * 4 == seq and nchunks % 2 == 0
    mesh = plsc.VectorSubcoreMesh(core_axis_name="c", subcore_axis_name="s")

    @functools.partial(
        pl.kernel,
        out_type=jax.ShapeDtypeStruct(
            (seq, _HIDDEN // 8, nbt, 8, _GRP), jnp.float32
        ),
        mesh=mesh,
        scratch_types=[
            pltpu.VMEM((4, _GRP), jnp.int32),
            pltpu.VMEM((4, _GRP), jnp.int32),
            pltpu.VMEM((4, _GRP, _HIDDEN), jnp.float32),
            pltpu.VMEM((4, _GRP, _HIDDEN), jnp.float32),
            pltpu.VMEM((2, _HIDDEN // 8, 8, _GRP), jnp.float32),
            pltpu.VMEM((2, _HIDDEN // 8, 8, _GRP), jnp.float32),
            pltpu.SemaphoreType.DMA,
            pltpu.SemaphoreType.DMA,
            pltpu.SemaphoreType.DMA,
            pltpu.SemaphoreType.DMA,
            pltpu.SemaphoreType.DMA,
            pltpu.SemaphoreType.DMA,
        ],
        compiler_params=pltpu.CompilerParams(
            use_tc_tiling_on_sc=False, needs_layout_passes=False,
            disable_bounds_checks=True,
        ),
    )
    def gather(table_hbm, ids_hbm, out_hbm,
               idx0, idx1, rows0, rows1, tp0, tp1,
               i0, i1, g0, g1, st0, st1):
        w = lax.axis_index("s") * _NC + lax.axis_index("c")
        idx_v = (idx0, idx1)
        rows_v = (rows0, rows1)
        tpb = (tp0, tp1)
        isem = (i0, i1)
        gsem = (g0, g1)
        stsem = (st0, st1)

        def fire_idx(c, b):
            pltpu.async_copy(
                ids_hbm.at[c // 2, w, pl.ds((4 * c) % 8, 4)],
                idx_v[b], isem[b],
            )

        def wait_idx(b):
            pltpu.make_async_copy(
                ids_hbm.at[0, 0, pl.ds(0, 4)], idx_v[b], isem[b]
            ).wait()

        def fire_gathers(b):
            for j in range(4):
                pltpu.async_copy(
                    table_hbm.at[idx_v[b].at[j]], rows_v[b].at[j], gsem[b]
                )

        def wait_gathers(b):
            for j in range(4):
                pltpu.make_async_copy(
                    table_hbm.at[pl.ds(0, _GRP)], rows_v[b].at[j], gsem[b]
                ).wait()

        def wait_pair_store(p):
            for u in range(2):
                pltpu.make_async_copy(
                    tpb[p].at[u], out_hbm.at[0, :, w], stsem[p]
                ).wait()

        iota16 = lax.iota(jnp.int32, _L)
        rows_c = [c0 * _L + iota16 for c0 in range(8)]
        cps = [(iota16 + k) & 15 for k in range(_L)]

        def transpose_pair(src4, j2, dst):
            @pl.loop(0, 2)
            def _(u):
                src = src4.at[2 * j2 + u]
                d = dst.at[u]

                @plsc.parallel_loop(0, 8)
                def _(tb8):
                    trow = tb8 * _L + iota16
                    for hb in range(4):
                        h0 = hb * _L
                        hvs = [h0 + cps[kk] for kk in range(_L)]
                        vecs = [
                            plsc.load_gather(src, [trow, hvs[kk]])
                            for kk in range(_L)
                        ]
                        for kk in range(_L):
                            hv = hvs[kk]
                            plsc.store_scatter(
                                d,
                                [lax.shift_right_logical(hv, 3), hv & 7,
                                 trow],
                                vecs[kk],
                            )

        def transpose_store(c, b):
            for p in range(2):
                if b == 0:
                    @pl.when(c > 0)
                    def _():
                        wait_pair_store(p)
                else:
                    wait_pair_store(p)
                transpose_pair(rows_v[b], p, tpb[p])
                s0 = 4 * c + 2 * p
                for u in range(2):
                    pltpu.async_copy(
                        tpb[p].at[u], out_hbm.at[s0 + u, :, w], stsem[p]
                    )

        fire_idx(0, 0)
        wait_idx(0)
        fire_gathers(0)
        fire_idx(1, 1)

        @pl.loop(0, nchunks, step=2)
        def _(c):
            wait_idx(1)
            fire_gathers(1)
            wait_gathers(0)

            @pl.when(c + 2 < nchunks)
            def _():
                fire_idx(c + 2, 0)

            transpose_store(c, 0)

            @pl.when(c + 2 < nchunks)
            def _():
                wait_idx(0)
                fire_gathers(0)

            wait_gathers(1)

            @pl.when(c + 3 < nchunks)
            def _():
                fire_idx(c + 3, 1)

            transpose_store(c + 1, 1)

        wait_pair_store(0)
        wait_pair_store(1)

    return gather


def _ws_body(ws_ref, out_ref):
    out_ref[...] = jnp.broadcast_to(ws_ref[...], out_ref.shape)


@functools.cache
def _make_ws_broadcast(bs, w):
    blk = 256
    assert bs % blk == 0
    return pl.pallas_call(
        _ws_body,
        grid=(bs // blk,),
        in_specs=[pl.BlockSpec((1, w, _HIDDEN), lambda i: (0, 0, 0))],
        out_specs=pl.BlockSpec((blk, w, _HIDDEN), lambda i: (i, 0, 0)),
        out_shape=jax.ShapeDtypeStruct((bs, w, _HIDDEN), jnp.float32),
    )


def kernel(input_ids, attention_mask, init_workspace, emb_table):
    bs, seq = input_ids.shape
    vocab = emb_table.shape[0]
    full = (vocab // _GRP) * _GRP
    tail2 = emb_table[full:].reshape((vocab - full) // 2, 2 * _HIDDEN)
    packed = _make_packer(vocab)(emb_table.T, tail2)
    table_lin = packed.reshape(packed.shape[0] * 2, _HIDDEN)
    ids4 = (
        input_ids.T.reshape(seq // 8, 8, bs // _GRP, _GRP)
        .transpose(0, 2, 1, 3)
    )
    out5 = _make_gather(bs, seq)(table_lin, ids4)
    embeddings = jnp.transpose(out5, (2, 4, 0, 1, 3)).reshape(bs, seq, _HIDDEN)
    workspace = _make_ws_broadcast(bs, init_workspace.shape[1])(init_workspace)
    return (workspace, embeddings)
